# Initial kernel scaffold; baseline (speedup 1.0000x reference)
import jax
import jax.numpy as jnp
from jax.experimental import pallas as pl


def kernel(robot_x, human_x, edge_index, edge_weight, wr_w0, wr_b0, wr_w1, wr_b1, wh_w0, wh_b0, wh_w1, wh_b1, dz_w, dz_b, dr_w, dr_b, dh_w, dh_b):
    raise NotImplementedError("write your pallas kernel here")


# speedup vs baseline: 185.7071x; 185.7071x over previous
import functools

import jax
import jax.numpy as jnp
from jax import lax
from jax.experimental import pallas as pl
from jax.experimental.pallas import tpu as pltpu
from jax.experimental.pallas import tpu_sc as plsc

B = 4
N = 12500
E = 400000
XD = 32
NPAD = 12800
GROUPS = 8
EPT = E // GROUPS
CE = 2000
NCHUNK = EPT // CE
SLICE = NPAD // GROUPS
CHUNK = 1280
TC_C = NPAD // CHUNK


def _sc_body(ei, ew, out, es, ed, ewv, v0, v1, v2, v3, rbuf, racc, degsl,
             bof, bif, stage, pub):
    cid = lax.axis_index("c")
    sid = lax.axis_index("s")
    bt_loc = sid // GROUPS
    g = sid % GROUPS
    bt = cid * 2 + bt_loc
    ebase = g * EPT
    sl0 = g * SLICE
    src_base = bt * 2 * E + ebase
    dst_base = (bt * 2 + 1) * E + ebase
    ew_base = bt * E + ebase

    def stg(j, k):
        return ((bt_loc * GROUPS + j) * 2 + k) * NPAD

    zeros16 = jnp.zeros((16,), jnp.float32)

    def zero4(i, _):
        ds = pl.ds(i * 16, 16)
        v0[ds] = zeros16
        v1[ds] = zeros16
        v2[ds] = zeros16
        v3[ds] = zeros16
        return 0

    lax.fori_loop(0, NPAD // 16, zero4, 0)

    def p1_chunk(ci, _):
        off = ci * CE
        pltpu.sync_copy(ei.at[pl.ds(src_base + off, CE)], es)
        pltpu.sync_copy(ei.at[pl.ds(dst_base + off, CE)], ed)
        pltpu.sync_copy(ew.at[pl.ds(ew_base + off, CE)], ewv)

        def p1_grp(i, _):
            ds = pl.ds(i * 16, 16)
            s = es[ds]
            d = ed[ds]
            w = ewv[ds]
            plsc.addupdate_scatter(v0, [s], w)
            plsc.addupdate_scatter(v1, [d], w)
            plsc.addupdate_scatter(v2, [s], w, mask=(d == 0))
            plsc.addupdate_scatter(v3, [d], w, mask=(s == 0))
            return 0

        lax.fori_loop(0, CE // 16, p1_grp, 0)
        return 0

    lax.fori_loop(0, NCHUNK, p1_chunk, 0)

    pltpu.sync_copy(v0, stage.at[pl.ds(stg(g, 0), NPAD)])
    pltpu.sync_copy(v1, stage.at[pl.ds(stg(g, 1), NPAD)])
    plsc.subcore_barrier()

    for j in range(GROUPS):
        dstbuf = degsl if j == 0 else rbuf
        for k in range(2):
            pltpu.sync_copy(stage.at[pl.ds(stg(j, k) + sl0, SLICE)],
                            dstbuf.at[pl.ds(k * SLICE, SLICE)])
        if j > 0:
            def radd(i, _):
                ds = pl.ds(i * 16, 16)
                degsl[ds] = degsl[ds] + rbuf[ds]
                return 0

            lax.fori_loop(0, 2 * SLICE // 16, radd, 0)

    plsc.subcore_barrier()

    pltpu.sync_copy(v2, stage.at[pl.ds(stg(g, 0), NPAD)])
    pltpu.sync_copy(v3, stage.at[pl.ds(stg(g, 1), NPAD)])
    plsc.subcore_barrier()

    for j in range(GROUPS):
        dstbuf = racc if j == 0 else rbuf
        for k in range(2):
            pltpu.sync_copy(stage.at[pl.ds(stg(j, k) + sl0, SLICE)],
                            dstbuf.at[pl.ds(k * SLICE, SLICE)])
        if j > 0:
            def raddb(i, _):
                ds = pl.ds(i * 16, 16)
                racc[ds] = racc[ds] + rbuf[ds]
                return 0

            lax.fori_loop(0, 2 * SLICE // 16, raddb, 0)

    def mk_b(i, _):
        ds = pl.ds(i * 16, 16)
        ds1 = pl.ds(SLICE + i * 16, 16)
        dego = degsl[ds]
        degi = degsl[ds1]
        rbuf[ds] = racc[ds] / jnp.where(dego > 0.0, dego, 1.0)
        rbuf[ds1] = racc[ds1] / jnp.where(degi > 0.0, degi, 1.0)
        return 0

    lax.fori_loop(0, SLICE // 16, mk_b, 0)

    pltpu.sync_copy(rbuf.at[pl.ds(0, SLICE)],
                    out.at[pl.ds((bt * 4 + 0) * NPAD + sl0, SLICE)])
    pltpu.sync_copy(rbuf.at[pl.ds(SLICE, SLICE)],
                    out.at[pl.ds((bt * 4 + 1) * NPAD + sl0, SLICE)])
    pltpu.sync_copy(rbuf.at[pl.ds(0, SLICE)],
                    pub.at[pl.ds(bt_loc * 2 * NPAD + sl0, SLICE)])
    pltpu.sync_copy(rbuf.at[pl.ds(SLICE, SLICE)],
                    pub.at[pl.ds((bt_loc * 2 + 1) * NPAD + sl0, SLICE)])
    plsc.subcore_barrier()

    pltpu.sync_copy(pub.at[pl.ds(bt_loc * 2 * NPAD, NPAD)], bof)
    pltpu.sync_copy(pub.at[pl.ds((bt_loc * 2 + 1) * NPAD, NPAD)], bif)

    def zero2(i, _):
        ds = pl.ds(i * 16, 16)
        v0[ds] = zeros16
        v1[ds] = zeros16
        return 0

    lax.fori_loop(0, NPAD // 16, zero2, 0)

    def p2_chunk(ci, _):
        off = ci * CE
        pltpu.sync_copy(ei.at[pl.ds(src_base + off, CE)], es)
        pltpu.sync_copy(ei.at[pl.ds(dst_base + off, CE)], ed)
        pltpu.sync_copy(ew.at[pl.ds(ew_base + off, CE)], ewv)

        def p2_grp(i, _):
            ds = pl.ds(i * 16, 16)
            s = es[ds]
            d = ed[ds]
            w = ewv[ds]
            tb = plsc.load_gather(bof, [d])
            plsc.addupdate_scatter(v0, [s], w * tb)
            ti = plsc.load_gather(bif, [s])
            plsc.addupdate_scatter(v1, [d], w * ti)
            return 0

        lax.fori_loop(0, CE // 16, p2_grp, 0)
        return 0

    lax.fori_loop(0, NCHUNK, p2_chunk, 0)

    pltpu.sync_copy(v0, stage.at[pl.ds(stg(g, 0), NPAD)])
    pltpu.sync_copy(v1, stage.at[pl.ds(stg(g, 1), NPAD)])
    plsc.subcore_barrier()

    for j in range(GROUPS):
        dstbuf = racc if j == 0 else rbuf
        for k in range(2):
            pltpu.sync_copy(stage.at[pl.ds(stg(j, k) + sl0, SLICE)],
                            dstbuf.at[pl.ds(k * SLICE, SLICE)])
        if j > 0:
            def radd2(i, _):
                ds = pl.ds(i * 16, 16)
                racc[ds] = racc[ds] + rbuf[ds]
                return 0

            lax.fori_loop(0, 2 * SLICE // 16, radd2, 0)

    def mk_a(i, _):
        ds = pl.ds(i * 16, 16)
        ds1 = pl.ds(SLICE + i * 16, 16)
        dego = degsl[ds]
        degi = degsl[ds1]
        rbuf[ds] = racc[ds] / jnp.where(dego > 0.0, dego, 1.0)
        rbuf[ds1] = racc[ds1] / jnp.where(degi > 0.0, degi, 1.0)
        return 0

    lax.fori_loop(0, SLICE // 16, mk_a, 0)

    pltpu.sync_copy(rbuf.at[pl.ds(0, SLICE)],
                    out.at[pl.ds((bt * 4 + 2) * NPAD + sl0, SLICE)])
    pltpu.sync_copy(rbuf.at[pl.ds(SLICE, SLICE)],
                    out.at[pl.ds((bt * 4 + 3) * NPAD + sl0, SLICE)])


_sc_coeffs = pl.kernel(
    _sc_body,
    out_type=jax.ShapeDtypeStruct((B * 4 * NPAD,), jnp.float32),
    mesh=plsc.VectorSubcoreMesh(core_axis_name="c", subcore_axis_name="s"),
    compiler_params=pltpu.CompilerParams(needs_layout_passes=False),
    scratch_types=[
        pltpu.VMEM((CE,), jnp.int32),
        pltpu.VMEM((CE,), jnp.int32),
        pltpu.VMEM((CE,), jnp.float32),
        pltpu.VMEM((NPAD,), jnp.float32),
        pltpu.VMEM((NPAD,), jnp.float32),
        pltpu.VMEM((NPAD,), jnp.float32),
        pltpu.VMEM((NPAD,), jnp.float32),
        pltpu.VMEM((2 * SLICE,), jnp.float32),
        pltpu.VMEM((2 * SLICE,), jnp.float32),
        pltpu.VMEM((2 * SLICE,), jnp.float32),
        pltpu.VMEM((NPAD,), jnp.float32),
        pltpu.VMEM((NPAD,), jnp.float32),
        pltpu.VMEM_SHARED((2 * GROUPS * 2 * NPAD,), jnp.float32),
        pltpu.VMEM_SHARED((2 * 2 * NPAD,), jnp.float32),
    ],
)


def _tc_body(hx, bv, rx, wr0, wrb0, wr1, wrb1, wh0, whb0, wh1, whb1,
             dzw, dzb, dhw, dhb, out, acc, vb0):
    c = pl.program_id(1)

    x = hx[0]
    h = jnp.maximum(
        jnp.dot(x, wh0[...], preferred_element_type=jnp.float32) + whb0[...],
        0.0)
    xh = jnp.maximum(
        jnp.dot(h, wh1[...], preferred_element_type=jnp.float32) + whb1[...],
        0.0)
    bvv = bv[0]
    part = jnp.dot(bvv, xh, preferred_element_type=jnp.float32)

    @pl.when(c == 0)
    def _():
        acc[...] = part
        vb0[...] = jnp.broadcast_to(bvv[:, 0:1], (4, XD))

    @pl.when(c > 0)
    def _():
        acc[...] = acc[...] + part

    @pl.when(c == TC_C - 1)
    def _():
        r = rx[0]
        h0 = jnp.maximum(
            jnp.dot(r, wr0[...], preferred_element_type=jnp.float32)
            + wrb0[...], 0.0)
        x0 = jnp.maximum(
            jnp.dot(h0, wr1[...], preferred_element_type=jnp.float32)
            + wrb1[...], 0.0)
        hz = jnp.maximum(whb0[...], 0.0)
        xh0 = jnp.maximum(
            jnp.dot(hz, wh1[...], preferred_element_type=jnp.float32)
            + whb1[...], 0.0)
        v = acc[...] + vb0[...] * (x0 - xh0)
        v1o = v[0:1]
        v1i = v[1:2]
        v2o = 2.0 * v[2:3] - x0
        v2i = 2.0 * v[3:4] - x0

        def gate(wref, bref):
            w = wref[...]
            pre = (jnp.dot(x0, w[0, 0, :XD, :], preferred_element_type=jnp.float32)
                   + jnp.dot(x0, w[1, 0, :XD, :], preferred_element_type=jnp.float32)
                   + jnp.dot(v1o, w[0, 1, :XD, :], preferred_element_type=jnp.float32)
                   + jnp.dot(v1i, w[1, 1, :XD, :], preferred_element_type=jnp.float32)
                   + jnp.dot(v2o, w[0, 2, :XD, :], preferred_element_type=jnp.float32)
                   + jnp.dot(v2i, w[1, 2, :XD, :], preferred_element_type=jnp.float32))
            return pre + bref[...]

        z = jax.nn.sigmoid(gate(dzw, dzb))
        ht = jnp.tanh(gate(dhw, dhb))
        out[0] = (1.0 - z) * ht


def _tc_call(hx_pad, bvecs, robot_x, wr_w0, wr_b0, wr_w1, wr_b1,
             wh_w0, wh_b0, wh_w1, wh_b1, dz_w, dz_b, dh_w, dh_b):
    full = lambda *shape: pl.BlockSpec(shape, lambda b, c: (0,) * len(shape))
    return pl.pallas_call(
        _tc_body,
        grid=(B, TC_C),
        in_specs=[
            pl.BlockSpec((1, CHUNK, 5), lambda b, c: (b, c, 0)),
            pl.BlockSpec((1, 4, CHUNK), lambda b, c: (b, 0, c)),
            pl.BlockSpec((1, 1, 9), lambda b, c: (b, 0, 0)),
            full(9, 64), full(1, 64), full(64, 32), full(1, 32),
            full(5, 64), full(1, 64), full(64, 32), full(1, 32),
            full(2, 3, 64, 32), full(1, 32),
            full(2, 3, 64, 32), full(1, 32),
        ],
        out_specs=pl.BlockSpec((1, 1, XD), lambda b, c: (b, 0, 0)),
        out_shape=jax.ShapeDtypeStruct((B, 1, XD), jnp.float32),
        scratch_shapes=[
            pltpu.VMEM((4, XD), jnp.float32),
            pltpu.VMEM((4, XD), jnp.float32),
        ],
    )(hx_pad, bvecs, robot_x, wr_w0, wr_b0, wr_w1, wr_b1,
      wh_w0, wh_b0, wh_w1, wh_b1, dz_w, dz_b, dh_w, dh_b)


@jax.jit
def kernel(robot_x, human_x, edge_index, edge_weight, wr_w0, wr_b0, wr_w1,
           wr_b1, wh_w0, wh_b0, wh_w1, wh_b1, dz_w, dz_b, dr_w, dr_b,
           dh_w, dh_b):
    del dr_w, dr_b
    bvecs = _sc_coeffs(edge_index.reshape(-1),
                       edge_weight.reshape(-1)).reshape(B, 4, NPAD)
    nh = human_x.shape[1]
    hx_pad = jnp.concatenate(
        [jnp.zeros((B, 1, 5), jnp.float32), human_x,
         jnp.zeros((B, NPAD - 1 - nh, 5), jnp.float32)], axis=1)
    res = _tc_call(
        hx_pad, bvecs, robot_x,
        wr_w0, wr_b0.reshape(1, 64), wr_w1, wr_b1.reshape(1, 32),
        wh_w0, wh_b0.reshape(1, 64), wh_w1, wh_b1.reshape(1, 32),
        dz_w, dz_b.reshape(1, 32), dh_w, dh_b.reshape(1, 32))
    return res.reshape(B, XD)


# speedup vs baseline: 268.4450x; 1.4455x over previous
import functools

import jax
import jax.numpy as jnp
from jax import lax
from jax.experimental import pallas as pl
from jax.experimental.pallas import tpu as pltpu
from jax.experimental.pallas import tpu_sc as plsc

B = 4
N = 12500
E = 400000
XD = 32
NPAD = 12800
GROUPS = 8
EPT = E // GROUPS
CE = 2000
NCHUNK = EPT // CE
SLICE = NPAD // GROUPS
CHUNK = 1280
TC_C = NPAD // CHUNK


def _sc_body(ei, ew, out, es0, ed0, ew0, es1, ed1, ew1, v0, v1, v2, v3,
             rbuf, racc, degsl, bof, bif, stage, pub, sem0, sem1):
    cid = lax.axis_index("c")
    sid = lax.axis_index("s")
    bt_loc = sid // GROUPS
    g = sid % GROUPS
    bt = cid * 2 + bt_loc
    ebase = g * EPT
    sl0 = g * SLICE
    src_base = bt * 2 * E + ebase
    dst_base = (bt * 2 + 1) * E + ebase
    ew_base = bt * E + ebase

    def stg(j, k):
        return ((bt_loc * GROUPS + j) * 2 + k) * NPAD

    bufs = ((es0, ed0, ew0), (es1, ed1, ew1))
    sems = (sem0, sem1)

    def edge_pass(process_grp):
        def start(ci, pi):
            off = ci * CE
            return (
                pltpu.async_copy(ei.at[pl.ds(src_base + off, CE)],
                                 bufs[pi][0], sems[pi]),
                pltpu.async_copy(ei.at[pl.ds(dst_base + off, CE)],
                                 bufs[pi][1], sems[pi]),
                pltpu.async_copy(ew.at[pl.ds(ew_base + off, CE)],
                                 bufs[pi][2], sems[pi]),
            )

        descs = [start(0, 0), None]
        for ci in range(NCHUNK):
            pi = ci % 2
            if ci + 1 < NCHUNK:
                descs[1 - pi] = start(ci + 1, 1 - pi)
            for dsc in descs[pi]:
                dsc.wait()
            process_grp(*bufs[pi])

    zeros16 = jnp.zeros((16,), jnp.float32)

    def zero4(i, _):
        ds = pl.ds(i * 16, 16)
        v0[ds] = zeros16
        v1[ds] = zeros16
        v2[ds] = zeros16
        v3[ds] = zeros16
        return 0

    lax.fori_loop(0, NPAD // 16, zero4, 0)

    def p1_proc(es, ed, ewv):
        def p1_grp(i, _):
            ds = pl.ds(i * 16, 16)
            s = es[ds]
            d = ed[ds]
            w = ewv[ds]
            plsc.addupdate_scatter(v0, [s], w)
            plsc.addupdate_scatter(v1, [d], w)
            plsc.addupdate_scatter(v2, [s], w, mask=(d == 0))
            plsc.addupdate_scatter(v3, [d], w, mask=(s == 0))
            return 0

        lax.fori_loop(0, CE // 16, p1_grp, 0)

    edge_pass(p1_proc)

    pltpu.sync_copy(v0, stage.at[pl.ds(stg(g, 0), NPAD)])
    pltpu.sync_copy(v1, stage.at[pl.ds(stg(g, 1), NPAD)])
    plsc.subcore_barrier()

    for j in range(GROUPS):
        dstbuf = degsl if j == 0 else rbuf
        for k in range(2):
            pltpu.sync_copy(stage.at[pl.ds(stg(j, k) + sl0, SLICE)],
                            dstbuf.at[pl.ds(k * SLICE, SLICE)])
        if j > 0:
            def radd(i, _):
                ds = pl.ds(i * 16, 16)
                degsl[ds] = degsl[ds] + rbuf[ds]
                return 0

            lax.fori_loop(0, 2 * SLICE // 16, radd, 0)

    plsc.subcore_barrier()

    pltpu.sync_copy(v2, stage.at[pl.ds(stg(g, 0), NPAD)])
    pltpu.sync_copy(v3, stage.at[pl.ds(stg(g, 1), NPAD)])
    plsc.subcore_barrier()

    for j in range(GROUPS):
        dstbuf = racc if j == 0 else rbuf
        for k in range(2):
            pltpu.sync_copy(stage.at[pl.ds(stg(j, k) + sl0, SLICE)],
                            dstbuf.at[pl.ds(k * SLICE, SLICE)])
        if j > 0:
            def raddb(i, _):
                ds = pl.ds(i * 16, 16)
                racc[ds] = racc[ds] + rbuf[ds]
                return 0

            lax.fori_loop(0, 2 * SLICE // 16, raddb, 0)

    def mk_b(i, _):
        ds = pl.ds(i * 16, 16)
        ds1 = pl.ds(SLICE + i * 16, 16)
        dego = degsl[ds]
        degi = degsl[ds1]
        rbuf[ds] = racc[ds] / jnp.where(dego > 0.0, dego, 1.0)
        rbuf[ds1] = racc[ds1] / jnp.where(degi > 0.0, degi, 1.0)
        return 0

    lax.fori_loop(0, SLICE // 16, mk_b, 0)

    pltpu.sync_copy(rbuf.at[pl.ds(0, SLICE)],
                    out.at[pl.ds((bt * 4 + 0) * NPAD + sl0, SLICE)])
    pltpu.sync_copy(rbuf.at[pl.ds(SLICE, SLICE)],
                    out.at[pl.ds((bt * 4 + 1) * NPAD + sl0, SLICE)])
    pltpu.sync_copy(rbuf.at[pl.ds(0, SLICE)],
                    pub.at[pl.ds(bt_loc * 2 * NPAD + sl0, SLICE)])
    pltpu.sync_copy(rbuf.at[pl.ds(SLICE, SLICE)],
                    pub.at[pl.ds((bt_loc * 2 + 1) * NPAD + sl0, SLICE)])
    plsc.subcore_barrier()

    pltpu.sync_copy(pub.at[pl.ds(bt_loc * 2 * NPAD, NPAD)], bof)
    pltpu.sync_copy(pub.at[pl.ds((bt_loc * 2 + 1) * NPAD, NPAD)], bif)

    def zero2(i, _):
        ds = pl.ds(i * 16, 16)
        v0[ds] = zeros16
        v1[ds] = zeros16
        return 0

    lax.fori_loop(0, NPAD // 16, zero2, 0)

    def p2_proc(es, ed, ewv):
        def p2_grp(i, _):
            ds = pl.ds(i * 16, 16)
            s = es[ds]
            d = ed[ds]
            w = ewv[ds]
            tb = plsc.load_gather(bof, [d])
            plsc.addupdate_scatter(v0, [s], w * tb)
            ti = plsc.load_gather(bif, [s])
            plsc.addupdate_scatter(v1, [d], w * ti)
            return 0

        lax.fori_loop(0, CE // 16, p2_grp, 0)

    edge_pass(p2_proc)

    pltpu.sync_copy(v0, stage.at[pl.ds(stg(g, 0), NPAD)])
    pltpu.sync_copy(v1, stage.at[pl.ds(stg(g, 1), NPAD)])
    plsc.subcore_barrier()

    for j in range(GROUPS):
        dstbuf = racc if j == 0 else rbuf
        for k in range(2):
            pltpu.sync_copy(stage.at[pl.ds(stg(j, k) + sl0, SLICE)],
                            dstbuf.at[pl.ds(k * SLICE, SLICE)])
        if j > 0:
            def radd2(i, _):
                ds = pl.ds(i * 16, 16)
                racc[ds] = racc[ds] + rbuf[ds]
                return 0

            lax.fori_loop(0, 2 * SLICE // 16, radd2, 0)

    def mk_a(i, _):
        ds = pl.ds(i * 16, 16)
        ds1 = pl.ds(SLICE + i * 16, 16)
        dego = degsl[ds]
        degi = degsl[ds1]
        rbuf[ds] = racc[ds] / jnp.where(dego > 0.0, dego, 1.0)
        rbuf[ds1] = racc[ds1] / jnp.where(degi > 0.0, degi, 1.0)
        return 0

    lax.fori_loop(0, SLICE // 16, mk_a, 0)

    pltpu.sync_copy(rbuf.at[pl.ds(0, SLICE)],
                    out.at[pl.ds((bt * 4 + 2) * NPAD + sl0, SLICE)])
    pltpu.sync_copy(rbuf.at[pl.ds(SLICE, SLICE)],
                    out.at[pl.ds((bt * 4 + 3) * NPAD + sl0, SLICE)])


_sc_coeffs = pl.kernel(
    _sc_body,
    out_type=jax.ShapeDtypeStruct((B * 4 * NPAD,), jnp.float32),
    mesh=plsc.VectorSubcoreMesh(core_axis_name="c", subcore_axis_name="s"),
    compiler_params=pltpu.CompilerParams(needs_layout_passes=False),
    scratch_types=[
        pltpu.VMEM((CE,), jnp.int32),
        pltpu.VMEM((CE,), jnp.int32),
        pltpu.VMEM((CE,), jnp.float32),
        pltpu.VMEM((CE,), jnp.int32),
        pltpu.VMEM((CE,), jnp.int32),
        pltpu.VMEM((CE,), jnp.float32),
        pltpu.VMEM((NPAD,), jnp.float32),
        pltpu.VMEM((NPAD,), jnp.float32),
        pltpu.VMEM((NPAD,), jnp.float32),
        pltpu.VMEM((NPAD,), jnp.float32),
        pltpu.VMEM((2 * SLICE,), jnp.float32),
        pltpu.VMEM((2 * SLICE,), jnp.float32),
        pltpu.VMEM((2 * SLICE,), jnp.float32),
        pltpu.VMEM((NPAD,), jnp.float32),
        pltpu.VMEM((NPAD,), jnp.float32),
        pltpu.VMEM_SHARED((2 * GROUPS * 2 * NPAD,), jnp.float32),
        pltpu.VMEM_SHARED((2 * 2 * NPAD,), jnp.float32),
        pltpu.SemaphoreType.DMA,
        pltpu.SemaphoreType.DMA,
    ],
)


def _tc_body(hx, bv, rx, wr0, wrb0, wr1, wrb1, wh0, whb0, wh1, whb1,
             dzw, dzb, dhw, dhb, out, acc, vb0):
    c = pl.program_id(1)

    x = hx[0]
    h = jnp.maximum(
        jnp.dot(x, wh0[...], preferred_element_type=jnp.float32) + whb0[...],
        0.0)
    xh = jnp.maximum(
        jnp.dot(h, wh1[...], preferred_element_type=jnp.float32) + whb1[...],
        0.0)
    bvv = bv[0]
    part = jnp.dot(bvv, xh, preferred_element_type=jnp.float32)

    @pl.when(c == 0)
    def _():
        acc[...] = part
        vb0[...] = jnp.broadcast_to(bvv[:, 0:1], (4, XD))

    @pl.when(c > 0)
    def _():
        acc[...] = acc[...] + part

    @pl.when(c == TC_C - 1)
    def _():
        r = rx[0]
        h0 = jnp.maximum(
            jnp.dot(r, wr0[...], preferred_element_type=jnp.float32)
            + wrb0[...], 0.0)
        x0 = jnp.maximum(
            jnp.dot(h0, wr1[...], preferred_element_type=jnp.float32)
            + wrb1[...], 0.0)
        hz = jnp.maximum(whb0[...], 0.0)
        xh0 = jnp.maximum(
            jnp.dot(hz, wh1[...], preferred_element_type=jnp.float32)
            + whb1[...], 0.0)
        v = acc[...] + vb0[...] * (x0 - xh0)
        v1o = v[0:1]
        v1i = v[1:2]
        v2o = 2.0 * v[2:3] - x0
        v2i = 2.0 * v[3:4] - x0

        def gate(wref, bref):
            w = wref[...]
            pre = (jnp.dot(x0, w[0, 0, :XD, :], preferred_element_type=jnp.float32)
                   + jnp.dot(x0, w[1, 0, :XD, :], preferred_element_type=jnp.float32)
                   + jnp.dot(v1o, w[0, 1, :XD, :], preferred_element_type=jnp.float32)
                   + jnp.dot(v1i, w[1, 1, :XD, :], preferred_element_type=jnp.float32)
                   + jnp.dot(v2o, w[0, 2, :XD, :], preferred_element_type=jnp.float32)
                   + jnp.dot(v2i, w[1, 2, :XD, :], preferred_element_type=jnp.float32))
            return pre + bref[...]

        z = jax.nn.sigmoid(gate(dzw, dzb))
        ht = jnp.tanh(gate(dhw, dhb))
        out[0] = (1.0 - z) * ht


def _tc_call(hx_pad, bvecs, robot_x, wr_w0, wr_b0, wr_w1, wr_b1,
             wh_w0, wh_b0, wh_w1, wh_b1, dz_w, dz_b, dh_w, dh_b):
    full = lambda *shape: pl.BlockSpec(shape, lambda b, c: (0,) * len(shape))
    return pl.pallas_call(
        _tc_body,
        grid=(B, TC_C),
        in_specs=[
            pl.BlockSpec((1, CHUNK, 5), lambda b, c: (b, c, 0)),
            pl.BlockSpec((1, 4, CHUNK), lambda b, c: (b, 0, c)),
            pl.BlockSpec((1, 1, 9), lambda b, c: (b, 0, 0)),
            full(9, 64), full(1, 64), full(64, 32), full(1, 32),
            full(5, 64), full(1, 64), full(64, 32), full(1, 32),
            full(2, 3, 64, 32), full(1, 32),
            full(2, 3, 64, 32), full(1, 32),
        ],
        out_specs=pl.BlockSpec((1, 1, XD), lambda b, c: (b, 0, 0)),
        out_shape=jax.ShapeDtypeStruct((B, 1, XD), jnp.float32),
        scratch_shapes=[
            pltpu.VMEM((4, XD), jnp.float32),
            pltpu.VMEM((4, XD), jnp.float32),
        ],
    )(hx_pad, bvecs, robot_x, wr_w0, wr_b0, wr_w1, wr_b1,
      wh_w0, wh_b0, wh_w1, wh_b1, dz_w, dz_b, dh_w, dh_b)


@jax.jit
def kernel(robot_x, human_x, edge_index, edge_weight, wr_w0, wr_b0, wr_w1,
           wr_b1, wh_w0, wh_b0, wh_w1, wh_b1, dz_w, dz_b, dr_w, dr_b,
           dh_w, dh_b):
    del dr_w, dr_b
    bvecs = _sc_coeffs(edge_index.reshape(-1),
                       edge_weight.reshape(-1)).reshape(B, 4, NPAD)
    nh = human_x.shape[1]
    hx_pad = jnp.concatenate(
        [jnp.zeros((B, 1, 5), jnp.float32), human_x,
         jnp.zeros((B, NPAD - 1 - nh, 5), jnp.float32)], axis=1)
    res = _tc_call(
        hx_pad, bvecs, robot_x,
        wr_w0, wr_b0.reshape(1, 64), wr_w1, wr_b1.reshape(1, 32),
        wh_w0, wh_b0.reshape(1, 64), wh_w1, wh_b1.reshape(1, 32),
        dz_w, dz_b.reshape(1, 32), dh_w, dh_b.reshape(1, 32))
    return res.reshape(B, XD)


# speedup vs baseline: 276.9068x; 1.0315x over previous
import functools

import jax
import jax.numpy as jnp
from jax import lax
from jax.experimental import pallas as pl
from jax.experimental.pallas import tpu as pltpu
from jax.experimental.pallas import tpu_sc as plsc

B = 4
N = 12500
E = 400000
XD = 32
NPAD = 12800
GROUPS = 8
EPT = E // GROUPS
CE = 2000
NCHUNK = EPT // CE
SLICE = NPAD // GROUPS
CHUNK = 1280
TC_C = NPAD // CHUNK


def _sc_body(ei, ew, out, es0, ed0, ew0, es1, ed1, ew1, v0, v1, v2, v3,
             rbuf, rbufb, racc, degsl, bof, bif, stage, pub, sem0, sem1):
    cid = lax.axis_index("c")
    sid = lax.axis_index("s")
    bt_loc = sid // GROUPS
    g = sid % GROUPS
    bt = cid * 2 + bt_loc
    ebase = g * EPT
    sl0 = g * SLICE
    src_base = bt * 2 * E + ebase
    dst_base = (bt * 2 + 1) * E + ebase
    ew_base = bt * E + ebase

    def stg(j, k):
        return ((bt_loc * GROUPS + j) * 2 + k) * NPAD

    bufs = ((es0, ed0, ew0), (es1, ed1, ew1))
    sems = (sem0, sem1)

    def edge_pass(process_grp):
        def start(ci, pi):
            off = ci * CE
            return (
                pltpu.async_copy(ei.at[pl.ds(src_base + off, CE)],
                                 bufs[pi][0], sems[pi]),
                pltpu.async_copy(ei.at[pl.ds(dst_base + off, CE)],
                                 bufs[pi][1], sems[pi]),
                pltpu.async_copy(ew.at[pl.ds(ew_base + off, CE)],
                                 bufs[pi][2], sems[pi]),
            )

        descs = [start(0, 0), None]
        for ci in range(NCHUNK):
            pi = ci % 2
            if ci + 1 < NCHUNK:
                descs[1 - pi] = start(ci + 1, 1 - pi)
            for dsc in descs[pi]:
                dsc.wait()
            process_grp(*bufs[pi])

    def reduce_round(acc_buf, lbuf_a, lbuf_b):
        def load(j, buf, sem):
            return [pltpu.async_copy(
                stage.at[pl.ds(stg(j, k) + sl0, SLICE)],
                buf.at[pl.ds(k * SLICE, SLICE)], sem) for k in (0, 1)]

        pend = load(0, acc_buf, sem0)
        nxt = load(1, lbuf_a, sem1)
        for d in pend:
            d.wait()
        for j in range(1, GROUPS):
            buf = lbuf_a if j % 2 == 1 else lbuf_b
            upcoming = None
            if j + 1 < GROUPS:
                nbuf = lbuf_a if (j + 1) % 2 == 1 else lbuf_b
                nsem = sem1 if (j + 1) % 2 == 1 else sem0
                upcoming = load(j + 1, nbuf, nsem)
            for d in nxt:
                d.wait()

            def radd(i, _):
                ds = pl.ds(i * 16, 16)
                acc_buf[ds] = acc_buf[ds] + buf[ds]
                return 0

            lax.fori_loop(0, 2 * SLICE // 16, radd, 0)
            nxt = upcoming

    zeros16 = jnp.zeros((16,), jnp.float32)

    def zero4(i, _):
        ds = pl.ds(i * 16, 16)
        v0[ds] = zeros16
        v1[ds] = zeros16
        v2[ds] = zeros16
        v3[ds] = zeros16
        return 0

    lax.fori_loop(0, NPAD // 16, zero4, 0)

    def p1_proc(es, ed, ewv):
        def p1_grp(i, _):
            ds = pl.ds(i * 16, 16)
            s = es[ds]
            d = ed[ds]
            w = ewv[ds]
            plsc.addupdate_scatter(v0, [s], w)
            plsc.addupdate_scatter(v1, [d], w)
            plsc.addupdate_scatter(v2, [s], w, mask=(d == 0))
            plsc.addupdate_scatter(v3, [d], w, mask=(s == 0))
            return 0

        lax.fori_loop(0, CE // 16, p1_grp, 0)

    edge_pass(p1_proc)

    da = pltpu.async_copy(v0, stage.at[pl.ds(stg(g, 0), NPAD)], sem0)
    db = pltpu.async_copy(v1, stage.at[pl.ds(stg(g, 1), NPAD)], sem1)
    da.wait()
    db.wait()
    plsc.subcore_barrier()

    reduce_round(degsl, rbuf, rbufb)

    plsc.subcore_barrier()

    da = pltpu.async_copy(v2, stage.at[pl.ds(stg(g, 0), NPAD)], sem0)
    db = pltpu.async_copy(v3, stage.at[pl.ds(stg(g, 1), NPAD)], sem1)
    da.wait()
    db.wait()
    plsc.subcore_barrier()

    reduce_round(racc, rbuf, rbufb)

    def mk_b(i, _):
        ds = pl.ds(i * 16, 16)
        ds1 = pl.ds(SLICE + i * 16, 16)
        dego = degsl[ds]
        degi = degsl[ds1]
        rbuf[ds] = racc[ds] / jnp.where(dego > 0.0, dego, 1.0)
        rbuf[ds1] = racc[ds1] / jnp.where(degi > 0.0, degi, 1.0)
        return 0

    lax.fori_loop(0, SLICE // 16, mk_b, 0)

    ds_ = [
        pltpu.async_copy(rbuf.at[pl.ds(0, SLICE)],
                         out.at[pl.ds((bt * 4 + 0) * NPAD + sl0, SLICE)],
                         sem0),
        pltpu.async_copy(rbuf.at[pl.ds(SLICE, SLICE)],
                         out.at[pl.ds((bt * 4 + 1) * NPAD + sl0, SLICE)],
                         sem0),
        pltpu.async_copy(rbuf.at[pl.ds(0, SLICE)],
                         pub.at[pl.ds(bt_loc * 2 * NPAD + sl0, SLICE)],
                         sem1),
        pltpu.async_copy(rbuf.at[pl.ds(SLICE, SLICE)],
                         pub.at[pl.ds((bt_loc * 2 + 1) * NPAD + sl0, SLICE)],
                         sem1),
    ]
    for d in ds_:
        d.wait()
    plsc.subcore_barrier()

    da = pltpu.async_copy(pub.at[pl.ds(bt_loc * 2 * NPAD, NPAD)], bof, sem0)
    db = pltpu.async_copy(pub.at[pl.ds((bt_loc * 2 + 1) * NPAD, NPAD)], bif,
                          sem1)
    da.wait()
    db.wait()

    def zero2(i, _):
        ds = pl.ds(i * 16, 16)
        v0[ds] = zeros16
        v1[ds] = zeros16
        return 0

    lax.fori_loop(0, NPAD // 16, zero2, 0)

    def p2_proc(es, ed, ewv):
        def p2_grp(i, _):
            ds = pl.ds(i * 16, 16)
            s = es[ds]
            d = ed[ds]
            w = ewv[ds]
            tb = plsc.load_gather(bof, [d])
            plsc.addupdate_scatter(v0, [s], w * tb)
            ti = plsc.load_gather(bif, [s])
            plsc.addupdate_scatter(v1, [d], w * ti)
            return 0

        lax.fori_loop(0, CE // 16, p2_grp, 0)

    edge_pass(p2_proc)

    da = pltpu.async_copy(v0, stage.at[pl.ds(stg(g, 0), NPAD)], sem0)
    db = pltpu.async_copy(v1, stage.at[pl.ds(stg(g, 1), NPAD)], sem1)
    da.wait()
    db.wait()
    plsc.subcore_barrier()

    reduce_round(racc, rbuf, rbufb)

    def mk_a(i, _):
        ds = pl.ds(i * 16, 16)
        ds1 = pl.ds(SLICE + i * 16, 16)
        dego = degsl[ds]
        degi = degsl[ds1]
        rbuf[ds] = racc[ds] / jnp.where(dego > 0.0, dego, 1.0)
        rbuf[ds1] = racc[ds1] / jnp.where(degi > 0.0, degi, 1.0)
        return 0

    lax.fori_loop(0, SLICE // 16, mk_a, 0)

    da = pltpu.async_copy(rbuf.at[pl.ds(0, SLICE)],
                          out.at[pl.ds((bt * 4 + 2) * NPAD + sl0, SLICE)],
                          sem0)
    db = pltpu.async_copy(rbuf.at[pl.ds(SLICE, SLICE)],
                          out.at[pl.ds((bt * 4 + 3) * NPAD + sl0, SLICE)],
                          sem1)
    da.wait()
    db.wait()


_sc_coeffs = pl.kernel(
    _sc_body,
    out_type=jax.ShapeDtypeStruct((B * 4 * NPAD,), jnp.float32),
    mesh=plsc.VectorSubcoreMesh(core_axis_name="c", subcore_axis_name="s"),
    compiler_params=pltpu.CompilerParams(needs_layout_passes=False),
    scratch_types=[
        pltpu.VMEM((CE,), jnp.int32),
        pltpu.VMEM((CE,), jnp.int32),
        pltpu.VMEM((CE,), jnp.float32),
        pltpu.VMEM((CE,), jnp.int32),
        pltpu.VMEM((CE,), jnp.int32),
        pltpu.VMEM((CE,), jnp.float32),
        pltpu.VMEM((NPAD,), jnp.float32),
        pltpu.VMEM((NPAD,), jnp.float32),
        pltpu.VMEM((NPAD,), jnp.float32),
        pltpu.VMEM((NPAD,), jnp.float32),
        pltpu.VMEM((2 * SLICE,), jnp.float32),
        pltpu.VMEM((2 * SLICE,), jnp.float32),
        pltpu.VMEM((2 * SLICE,), jnp.float32),
        pltpu.VMEM((2 * SLICE,), jnp.float32),
        pltpu.VMEM((NPAD,), jnp.float32),
        pltpu.VMEM((NPAD,), jnp.float32),
        pltpu.VMEM_SHARED((2 * GROUPS * 2 * NPAD,), jnp.float32),
        pltpu.VMEM_SHARED((2 * 2 * NPAD,), jnp.float32),
        pltpu.SemaphoreType.DMA,
        pltpu.SemaphoreType.DMA,
    ],
)


def _tc_body(hx, bv, rx, wr0, wrb0, wr1, wrb1, wh0, whb0, wh1, whb1,
             dzw, dzb, dhw, dhb, out, acc, vb0):
    c = pl.program_id(1)

    x = hx[0]
    h = jnp.maximum(
        jnp.dot(x, wh0[...], preferred_element_type=jnp.float32) + whb0[...],
        0.0)
    xh = jnp.maximum(
        jnp.dot(h, wh1[...], preferred_element_type=jnp.float32) + whb1[...],
        0.0)
    bvv = bv[0]
    part = jnp.dot(bvv, xh, preferred_element_type=jnp.float32)

    @pl.when(c == 0)
    def _():
        acc[...] = part
        vb0[...] = jnp.broadcast_to(bvv[:, 0:1], (4, XD))

    @pl.when(c > 0)
    def _():
        acc[...] = acc[...] + part

    @pl.when(c == TC_C - 1)
    def _():
        r = rx[0]
        h0 = jnp.maximum(
            jnp.dot(r, wr0[...], preferred_element_type=jnp.float32)
            + wrb0[...], 0.0)
        x0 = jnp.maximum(
            jnp.dot(h0, wr1[...], preferred_element_type=jnp.float32)
            + wrb1[...], 0.0)
        hz = jnp.maximum(whb0[...], 0.0)
        xh0 = jnp.maximum(
            jnp.dot(hz, wh1[...], preferred_element_type=jnp.float32)
            + whb1[...], 0.0)
        v = acc[...] + vb0[...] * (x0 - xh0)
        v1o = v[0:1]
        v1i = v[1:2]
        v2o = 2.0 * v[2:3] - x0
        v2i = 2.0 * v[3:4] - x0

        def gate(wref, bref):
            w = wref[...]
            pre = (jnp.dot(x0, w[0, 0, :XD, :], preferred_element_type=jnp.float32)
                   + jnp.dot(x0, w[1, 0, :XD, :], preferred_element_type=jnp.float32)
                   + jnp.dot(v1o, w[0, 1, :XD, :], preferred_element_type=jnp.float32)
                   + jnp.dot(v1i, w[1, 1, :XD, :], preferred_element_type=jnp.float32)
                   + jnp.dot(v2o, w[0, 2, :XD, :], preferred_element_type=jnp.float32)
                   + jnp.dot(v2i, w[1, 2, :XD, :], preferred_element_type=jnp.float32))
            return pre + bref[...]

        z = jax.nn.sigmoid(gate(dzw, dzb))
        ht = jnp.tanh(gate(dhw, dhb))
        out[0] = (1.0 - z) * ht


def _tc_call(hx_pad, bvecs, robot_x, wr_w0, wr_b0, wr_w1, wr_b1,
             wh_w0, wh_b0, wh_w1, wh_b1, dz_w, dz_b, dh_w, dh_b):
    full = lambda *shape: pl.BlockSpec(shape, lambda b, c: (0,) * len(shape))
    return pl.pallas_call(
        _tc_body,
        grid=(B, TC_C),
        in_specs=[
            pl.BlockSpec((1, CHUNK, 5), lambda b, c: (b, c, 0)),
            pl.BlockSpec((1, 4, CHUNK), lambda b, c: (b, 0, c)),
            pl.BlockSpec((1, 1, 9), lambda b, c: (b, 0, 0)),
            full(9, 64), full(1, 64), full(64, 32), full(1, 32),
            full(5, 64), full(1, 64), full(64, 32), full(1, 32),
            full(2, 3, 64, 32), full(1, 32),
            full(2, 3, 64, 32), full(1, 32),
        ],
        out_specs=pl.BlockSpec((1, 1, XD), lambda b, c: (b, 0, 0)),
        out_shape=jax.ShapeDtypeStruct((B, 1, XD), jnp.float32),
        scratch_shapes=[
            pltpu.VMEM((4, XD), jnp.float32),
            pltpu.VMEM((4, XD), jnp.float32),
        ],
    )(hx_pad, bvecs, robot_x, wr_w0, wr_b0, wr_w1, wr_b1,
      wh_w0, wh_b0, wh_w1, wh_b1, dz_w, dz_b, dh_w, dh_b)


@jax.jit
def kernel(robot_x, human_x, edge_index, edge_weight, wr_w0, wr_b0, wr_w1,
           wr_b1, wh_w0, wh_b0, wh_w1, wh_b1, dz_w, dz_b, dr_w, dr_b,
           dh_w, dh_b):
    del dr_w, dr_b
    bvecs = _sc_coeffs(edge_index.reshape(-1),
                       edge_weight.reshape(-1)).reshape(B, 4, NPAD)
    nh = human_x.shape[1]
    hx_pad = jnp.concatenate(
        [jnp.zeros((B, 1, 5), jnp.float32), human_x,
         jnp.zeros((B, NPAD - 1 - nh, 5), jnp.float32)], axis=1)
    res = _tc_call(
        hx_pad, bvecs, robot_x,
        wr_w0, wr_b0.reshape(1, 64), wr_w1, wr_b1.reshape(1, 32),
        wh_w0, wh_b0.reshape(1, 64), wh_w1, wh_b1.reshape(1, 32),
        dz_w, dz_b.reshape(1, 32), dh_w, dh_b.reshape(1, 32))
    return res.reshape(B, XD)


# speedup vs baseline: 306.7002x; 1.1076x over previous
import functools

import jax
import jax.numpy as jnp
from jax import lax
from jax.experimental import pallas as pl
from jax.experimental.pallas import tpu as pltpu
from jax.experimental.pallas import tpu_sc as plsc

B = 4
N = 12500
E = 400000
XD = 32
NPAD = 12800
GROUPS = 8
EPT = E // GROUPS
CE = 2000
NCHUNK = EPT // CE
SLICE = NPAD // GROUPS
CHUNK = 2560
TC_C = NPAD // CHUNK


def _sc_body(ei, ew, out, es0, ed0, ew0, es1, ed1, ew1, v0, v1, v2, v3,
             rbuf, rbufb, racc, degsl, bof, bif, stage, pub, sem0, sem1):
    cid = lax.axis_index("c")
    sid = lax.axis_index("s")
    bt_loc = sid // GROUPS
    g = sid % GROUPS
    bt = cid * 2 + bt_loc
    ebase = g * EPT
    sl0 = g * SLICE
    src_base = bt * 2 * E + ebase
    dst_base = (bt * 2 + 1) * E + ebase
    ew_base = bt * E + ebase

    def stg(j, k):
        return ((bt_loc * GROUPS + j) * 2 + k) * NPAD

    bufs = ((es0, ed0, ew0), (es1, ed1, ew1))
    sems = (sem0, sem1)

    def edge_pass(process_grp):
        def start(ci, pi):
            off = ci * CE
            return (
                pltpu.async_copy(ei.at[pl.ds(src_base + off, CE)],
                                 bufs[pi][0], sems[pi]),
                pltpu.async_copy(ei.at[pl.ds(dst_base + off, CE)],
                                 bufs[pi][1], sems[pi]),
                pltpu.async_copy(ew.at[pl.ds(ew_base + off, CE)],
                                 bufs[pi][2], sems[pi]),
            )

        descs = [start(0, 0), None]
        for ci in range(NCHUNK):
            pi = ci % 2
            if ci + 1 < NCHUNK:
                descs[1 - pi] = start(ci + 1, 1 - pi)
            for dsc in descs[pi]:
                dsc.wait()
            process_grp(*bufs[pi])

    def reduce_round(acc_buf, lbuf_a, lbuf_b):
        def load(j, buf, sem):
            return [pltpu.async_copy(
                stage.at[pl.ds(stg(j, k) + sl0, SLICE)],
                buf.at[pl.ds(k * SLICE, SLICE)], sem) for k in (0, 1)]

        pend = load(0, acc_buf, sem0)
        nxt = load(1, lbuf_a, sem1)
        for d in pend:
            d.wait()
        for j in range(1, GROUPS):
            buf = lbuf_a if j % 2 == 1 else lbuf_b
            upcoming = None
            if j + 1 < GROUPS:
                nbuf = lbuf_a if (j + 1) % 2 == 1 else lbuf_b
                nsem = sem1 if (j + 1) % 2 == 1 else sem0
                upcoming = load(j + 1, nbuf, nsem)
            for d in nxt:
                d.wait()

            def radd(i, _):
                ds = pl.ds(i * 16, 16)
                acc_buf[ds] = acc_buf[ds] + buf[ds]
                return 0

            lax.fori_loop(0, 2 * SLICE // 16, radd, 0)
            nxt = upcoming

    zeros16 = jnp.zeros((16,), jnp.float32)

    def zero4(i, _):
        ds = pl.ds(i * 16, 16)
        v0[ds] = zeros16
        v1[ds] = zeros16
        v2[ds] = zeros16
        v3[ds] = zeros16
        return 0

    lax.fori_loop(0, NPAD // 16, zero4, 0)

    def p1_proc(es, ed, ewv):
        def p1_grp(i, _):
            ds = pl.ds(i * 16, 16)
            s = es[ds]
            d = ed[ds]
            w = ewv[ds]
            plsc.addupdate_scatter(v0, [s], w)
            plsc.addupdate_scatter(v1, [d], w)
            plsc.addupdate_scatter(v2, [s], w, mask=(d == 0))
            plsc.addupdate_scatter(v3, [d], w, mask=(s == 0))
            return 0

        lax.fori_loop(0, CE // 16, p1_grp, 0)

    edge_pass(p1_proc)

    da = pltpu.async_copy(v0, stage.at[pl.ds(stg(g, 0), NPAD)], sem0)
    db = pltpu.async_copy(v1, stage.at[pl.ds(stg(g, 1), NPAD)], sem1)
    da.wait()
    db.wait()
    plsc.subcore_barrier()

    reduce_round(degsl, rbuf, rbufb)

    plsc.subcore_barrier()

    da = pltpu.async_copy(v2, stage.at[pl.ds(stg(g, 0), NPAD)], sem0)
    db = pltpu.async_copy(v3, stage.at[pl.ds(stg(g, 1), NPAD)], sem1)
    da.wait()
    db.wait()
    plsc.subcore_barrier()

    reduce_round(racc, rbuf, rbufb)

    def mk_b(i, _):
        ds = pl.ds(i * 16, 16)
        ds1 = pl.ds(SLICE + i * 16, 16)
        dego = degsl[ds]
        degi = degsl[ds1]
        rbuf[ds] = racc[ds] / jnp.where(dego > 0.0, dego, 1.0)
        rbuf[ds1] = racc[ds1] / jnp.where(degi > 0.0, degi, 1.0)
        return 0

    lax.fori_loop(0, SLICE // 16, mk_b, 0)

    ds_ = [
        pltpu.async_copy(rbuf.at[pl.ds(0, SLICE)],
                         out.at[pl.ds((bt * 4 + 0) * NPAD + sl0, SLICE)],
                         sem0),
        pltpu.async_copy(rbuf.at[pl.ds(SLICE, SLICE)],
                         out.at[pl.ds((bt * 4 + 1) * NPAD + sl0, SLICE)],
                         sem0),
        pltpu.async_copy(rbuf.at[pl.ds(0, SLICE)],
                         pub.at[pl.ds(bt_loc * 2 * NPAD + sl0, SLICE)],
                         sem1),
        pltpu.async_copy(rbuf.at[pl.ds(SLICE, SLICE)],
                         pub.at[pl.ds((bt_loc * 2 + 1) * NPAD + sl0, SLICE)],
                         sem1),
    ]
    for d in ds_:
        d.wait()
    plsc.subcore_barrier()

    da = pltpu.async_copy(pub.at[pl.ds(bt_loc * 2 * NPAD, NPAD)], bof, sem0)
    db = pltpu.async_copy(pub.at[pl.ds((bt_loc * 2 + 1) * NPAD, NPAD)], bif,
                          sem1)
    da.wait()
    db.wait()

    def zero2(i, _):
        ds = pl.ds(i * 16, 16)
        v0[ds] = zeros16
        v1[ds] = zeros16
        return 0

    lax.fori_loop(0, NPAD // 16, zero2, 0)

    def p2_proc(es, ed, ewv):
        def p2_grp(i, _):
            ds = pl.ds(i * 16, 16)
            s = es[ds]
            d = ed[ds]
            w = ewv[ds]
            tb = plsc.load_gather(bof, [d])
            plsc.addupdate_scatter(v0, [s], w * tb)
            ti = plsc.load_gather(bif, [s])
            plsc.addupdate_scatter(v1, [d], w * ti)
            return 0

        lax.fori_loop(0, CE // 16, p2_grp, 0)

    edge_pass(p2_proc)

    da = pltpu.async_copy(v0, stage.at[pl.ds(stg(g, 0), NPAD)], sem0)
    db = pltpu.async_copy(v1, stage.at[pl.ds(stg(g, 1), NPAD)], sem1)
    da.wait()
    db.wait()
    plsc.subcore_barrier()

    reduce_round(racc, rbuf, rbufb)

    def mk_a(i, _):
        ds = pl.ds(i * 16, 16)
        ds1 = pl.ds(SLICE + i * 16, 16)
        dego = degsl[ds]
        degi = degsl[ds1]
        rbuf[ds] = racc[ds] / jnp.where(dego > 0.0, dego, 1.0)
        rbuf[ds1] = racc[ds1] / jnp.where(degi > 0.0, degi, 1.0)
        return 0

    lax.fori_loop(0, SLICE // 16, mk_a, 0)

    da = pltpu.async_copy(rbuf.at[pl.ds(0, SLICE)],
                          out.at[pl.ds((bt * 4 + 2) * NPAD + sl0, SLICE)],
                          sem0)
    db = pltpu.async_copy(rbuf.at[pl.ds(SLICE, SLICE)],
                          out.at[pl.ds((bt * 4 + 3) * NPAD + sl0, SLICE)],
                          sem1)
    da.wait()
    db.wait()


_sc_coeffs = pl.kernel(
    _sc_body,
    out_type=jax.ShapeDtypeStruct((B * 4 * NPAD,), jnp.float32),
    mesh=plsc.VectorSubcoreMesh(core_axis_name="c", subcore_axis_name="s"),
    compiler_params=pltpu.CompilerParams(needs_layout_passes=False),
    scratch_types=[
        pltpu.VMEM((CE,), jnp.int32),
        pltpu.VMEM((CE,), jnp.int32),
        pltpu.VMEM((CE,), jnp.float32),
        pltpu.VMEM((CE,), jnp.int32),
        pltpu.VMEM((CE,), jnp.int32),
        pltpu.VMEM((CE,), jnp.float32),
        pltpu.VMEM((NPAD,), jnp.float32),
        pltpu.VMEM((NPAD,), jnp.float32),
        pltpu.VMEM((NPAD,), jnp.float32),
        pltpu.VMEM((NPAD,), jnp.float32),
        pltpu.VMEM((2 * SLICE,), jnp.float32),
        pltpu.VMEM((2 * SLICE,), jnp.float32),
        pltpu.VMEM((2 * SLICE,), jnp.float32),
        pltpu.VMEM((2 * SLICE,), jnp.float32),
        pltpu.VMEM((NPAD,), jnp.float32),
        pltpu.VMEM((NPAD,), jnp.float32),
        pltpu.VMEM_SHARED((2 * GROUPS * 2 * NPAD,), jnp.float32),
        pltpu.VMEM_SHARED((2 * 2 * NPAD,), jnp.float32),
        pltpu.SemaphoreType.DMA,
        pltpu.SemaphoreType.DMA,
    ],
)


def _tc_body(hx, bv, rx, wr0, wrb0, wr1, wrb1, wh0, whb0, wh1, whb1,
             dzw, dzb, dhw, dhb, out, acc, vb0):
    c = pl.program_id(0)

    x = hx[...].reshape(B * CHUNK, 5)
    h = jnp.maximum(
        jnp.dot(x, wh0[...], preferred_element_type=jnp.float32) + whb0[...],
        0.0)
    xh = jnp.maximum(
        jnp.dot(h, wh1[...], preferred_element_type=jnp.float32) + whb1[...],
        0.0)
    bvv = bv[...]

    @pl.when(c == 0)
    def _():
        vb0[...] = jnp.broadcast_to(bvv[:, :, 0:1], (B, 4, XD))

    for b in range(B):
        part = jnp.dot(bvv[b], xh[b * CHUNK:(b + 1) * CHUNK, :],
                       preferred_element_type=jnp.float32)

        @pl.when(c == 0)
        def _():
            acc[b] = part

        @pl.when(c > 0)
        def _():
            acc[b] = acc[b] + part

    @pl.when(c == TC_C - 1)
    def _():
        r = rx[...].reshape(B, 9)
        h0 = jnp.maximum(
            jnp.dot(r, wr0[...], preferred_element_type=jnp.float32)
            + wrb0[...], 0.0)
        x0 = jnp.maximum(
            jnp.dot(h0, wr1[...], preferred_element_type=jnp.float32)
            + wrb1[...], 0.0)
        hz = jnp.maximum(whb0[...], 0.0)
        xh0 = jnp.maximum(
            jnp.dot(hz, wh1[...], preferred_element_type=jnp.float32)
            + whb1[...], 0.0)
        corr = x0 - xh0
        v = acc[...] + vb0[...] * corr.reshape(B, 1, XD)
        v1o = v[:, 0, :]
        v1i = v[:, 1, :]
        v2o = 2.0 * v[:, 2, :] - x0
        v2i = 2.0 * v[:, 3, :] - x0

        def gate(wref, bref):
            w = wref[...]
            pre = (jnp.dot(x0, w[0, 0, :XD, :], preferred_element_type=jnp.float32)
                   + jnp.dot(x0, w[1, 0, :XD, :], preferred_element_type=jnp.float32)
                   + jnp.dot(v1o, w[0, 1, :XD, :], preferred_element_type=jnp.float32)
                   + jnp.dot(v1i, w[1, 1, :XD, :], preferred_element_type=jnp.float32)
                   + jnp.dot(v2o, w[0, 2, :XD, :], preferred_element_type=jnp.float32)
                   + jnp.dot(v2i, w[1, 2, :XD, :], preferred_element_type=jnp.float32))
            return pre + bref[...]

        z = jax.nn.sigmoid(gate(dzw, dzb))
        ht = jnp.tanh(gate(dhw, dhb))
        out[...] = ((1.0 - z) * ht).reshape(B, 1, XD)


def _tc_call(hx_pad, bvecs, robot_x, wr_w0, wr_b0, wr_w1, wr_b1,
             wh_w0, wh_b0, wh_w1, wh_b1, dz_w, dz_b, dh_w, dh_b):
    full = lambda *shape: pl.BlockSpec(shape, lambda c: (0,) * len(shape))
    return pl.pallas_call(
        _tc_body,
        grid=(TC_C,),
        in_specs=[
            pl.BlockSpec((B, CHUNK, 5), lambda c: (0, c, 0)),
            pl.BlockSpec((B, 4, CHUNK), lambda c: (0, 0, c)),
            full(B, 1, 9),
            full(9, 64), full(1, 64), full(64, 32), full(1, 32),
            full(5, 64), full(1, 64), full(64, 32), full(1, 32),
            full(2, 3, 64, 32), full(1, 32),
            full(2, 3, 64, 32), full(1, 32),
        ],
        out_specs=pl.BlockSpec((B, 1, XD), lambda c: (0, 0, 0)),
        out_shape=jax.ShapeDtypeStruct((B, 1, XD), jnp.float32),
        scratch_shapes=[
            pltpu.VMEM((B, 4, XD), jnp.float32),
            pltpu.VMEM((B, 4, XD), jnp.float32),
        ],
    )(hx_pad, bvecs, robot_x, wr_w0, wr_b0, wr_w1, wr_b1,
      wh_w0, wh_b0, wh_w1, wh_b1, dz_w, dz_b, dh_w, dh_b)


@jax.jit
def kernel(robot_x, human_x, edge_index, edge_weight, wr_w0, wr_b0, wr_w1,
           wr_b1, wh_w0, wh_b0, wh_w1, wh_b1, dz_w, dz_b, dr_w, dr_b,
           dh_w, dh_b):
    del dr_w, dr_b
    bvecs = _sc_coeffs(edge_index.reshape(-1),
                       edge_weight.reshape(-1)).reshape(B, 4, NPAD)
    nh = human_x.shape[1]
    hx_pad = jnp.concatenate(
        [jnp.zeros((B, 1, 5), jnp.float32), human_x,
         jnp.zeros((B, NPAD - 1 - nh, 5), jnp.float32)], axis=1)
    res = _tc_call(
        hx_pad, bvecs, robot_x,
        wr_w0, wr_b0.reshape(1, 64), wr_w1, wr_b1.reshape(1, 32),
        wh_w0, wh_b0.reshape(1, 64), wh_w1, wh_b1.reshape(1, 32),
        dz_w, dz_b.reshape(1, 32), dh_w, dh_b.reshape(1, 32))
    return res.reshape(B, XD)


# speedup vs baseline: 385.0436x; 1.2554x over previous
import functools

import jax
import jax.numpy as jnp
from jax import lax
from jax.experimental import pallas as pl
from jax.experimental.pallas import tpu as pltpu
from jax.experimental.pallas import tpu_sc as plsc

B = 4
N = 12500
E = 400000
XD = 32
NPAD = 12800
GROUPS = 8
EPT = E // GROUPS
CE = 2000
NCHUNK = EPT // CE
SLICE = NPAD // GROUPS
CHUNK = 2560
TC_C = NPAD // CHUNK


def _sc_body(ei, ew, out, es0, ed0, ew0, es1, ed1, ew1, v0, v1, v2, v3,
             rbuf, rbufb, racc, degsl, bof, bif, stage, pub, sem0, sem1):
    cid = lax.axis_index("c")
    sid = lax.axis_index("s")
    bt_loc = sid // GROUPS
    g = sid % GROUPS
    bt = cid * 2 + bt_loc
    ebase = g * EPT
    sl0 = g * SLICE
    src_base = bt * 2 * E + ebase
    dst_base = (bt * 2 + 1) * E + ebase
    ew_base = bt * E + ebase

    def stg(j, k):
        return ((bt_loc * GROUPS + j) * 2 + k) * NPAD

    bufs = ((es0, ed0, ew0), (es1, ed1, ew1))
    sems = (sem0, sem1)

    def edge_pass(process_grp):
        def start(ci, pi):
            off = ci * CE
            return (
                pltpu.async_copy(ei.at[pl.ds(src_base + off, CE)],
                                 bufs[pi][0], sems[pi]),
                pltpu.async_copy(ei.at[pl.ds(dst_base + off, CE)],
                                 bufs[pi][1], sems[pi]),
                pltpu.async_copy(ew.at[pl.ds(ew_base + off, CE)],
                                 bufs[pi][2], sems[pi]),
            )

        descs = [start(0, 0), None]
        for ci in range(NCHUNK):
            pi = ci % 2
            if ci + 1 < NCHUNK:
                descs[1 - pi] = start(ci + 1, 1 - pi)
            for dsc in descs[pi]:
                dsc.wait()
            process_grp(*bufs[pi])

    def reduce_round(acc_buf, lbuf_a, lbuf_b):
        def load(j, buf, sem):
            return [pltpu.async_copy(
                stage.at[pl.ds(stg(j, k) + sl0, SLICE)],
                buf.at[pl.ds(k * SLICE, SLICE)], sem) for k in (0, 1)]

        pend = load(0, acc_buf, sem0)
        nxt = load(1, lbuf_a, sem1)
        for d in pend:
            d.wait()
        for j in range(1, GROUPS):
            buf = lbuf_a if j % 2 == 1 else lbuf_b
            upcoming = None
            if j + 1 < GROUPS:
                nbuf = lbuf_a if (j + 1) % 2 == 1 else lbuf_b
                nsem = sem1 if (j + 1) % 2 == 1 else sem0
                upcoming = load(j + 1, nbuf, nsem)
            for d in nxt:
                d.wait()

            def radd(i, _):
                ds = pl.ds(i * 16, 16)
                acc_buf[ds] = acc_buf[ds] + buf[ds]
                return 0

            lax.fori_loop(0, 2 * SLICE // 16, radd, 0, unroll=4)
            nxt = upcoming

    zeros16 = jnp.zeros((16,), jnp.float32)

    def zero4(i, _):
        ds = pl.ds(i * 16, 16)
        v0[ds] = zeros16
        v1[ds] = zeros16
        v2[ds] = zeros16
        v3[ds] = zeros16
        return 0

    lax.fori_loop(0, NPAD // 16, zero4, 0, unroll=4)

    def p1_proc(es, ed, ewv):
        @plsc.parallel_loop(0, CE // 16, unroll=4)
        def p1_grp(i):
            ds = pl.ds(i * 16, 16)
            s = es[ds]
            d = ed[ds]
            w = ewv[ds]
            plsc.addupdate_scatter(v0, [s], w)
            plsc.addupdate_scatter(v1, [d], w)
            plsc.addupdate_scatter(v2, [s], w, mask=(d == 0))
            plsc.addupdate_scatter(v3, [d], w, mask=(s == 0))

    edge_pass(p1_proc)

    da = pltpu.async_copy(v0, stage.at[pl.ds(stg(g, 0), NPAD)], sem0)
    db = pltpu.async_copy(v1, stage.at[pl.ds(stg(g, 1), NPAD)], sem1)
    da.wait()
    db.wait()
    plsc.subcore_barrier()

    reduce_round(degsl, rbuf, rbufb)

    plsc.subcore_barrier()

    da = pltpu.async_copy(v2, stage.at[pl.ds(stg(g, 0), NPAD)], sem0)
    db = pltpu.async_copy(v3, stage.at[pl.ds(stg(g, 1), NPAD)], sem1)
    da.wait()
    db.wait()
    plsc.subcore_barrier()

    reduce_round(racc, rbuf, rbufb)

    def mk_b(i, _):
        ds = pl.ds(i * 16, 16)
        ds1 = pl.ds(SLICE + i * 16, 16)
        dego = degsl[ds]
        degi = degsl[ds1]
        rbuf[ds] = racc[ds] / jnp.where(dego > 0.0, dego, 1.0)
        rbuf[ds1] = racc[ds1] / jnp.where(degi > 0.0, degi, 1.0)
        return 0

    lax.fori_loop(0, SLICE // 16, mk_b, 0, unroll=2)

    ds_ = [
        pltpu.async_copy(rbuf.at[pl.ds(0, SLICE)],
                         out.at[pl.ds((bt * 4 + 0) * NPAD + sl0, SLICE)],
                         sem0),
        pltpu.async_copy(rbuf.at[pl.ds(SLICE, SLICE)],
                         out.at[pl.ds((bt * 4 + 1) * NPAD + sl0, SLICE)],
                         sem0),
        pltpu.async_copy(rbuf.at[pl.ds(0, SLICE)],
                         pub.at[pl.ds(bt_loc * 2 * NPAD + sl0, SLICE)],
                         sem1),
        pltpu.async_copy(rbuf.at[pl.ds(SLICE, SLICE)],
                         pub.at[pl.ds((bt_loc * 2 + 1) * NPAD + sl0, SLICE)],
                         sem1),
    ]
    for d in ds_:
        d.wait()
    plsc.subcore_barrier()

    da = pltpu.async_copy(pub.at[pl.ds(bt_loc * 2 * NPAD, NPAD)], bof, sem0)
    db = pltpu.async_copy(pub.at[pl.ds((bt_loc * 2 + 1) * NPAD, NPAD)], bif,
                          sem1)
    da.wait()
    db.wait()

    def zero2(i, _):
        ds = pl.ds(i * 16, 16)
        v0[ds] = zeros16
        v1[ds] = zeros16
        return 0

    lax.fori_loop(0, NPAD // 16, zero2, 0, unroll=4)

    def p2_proc(es, ed, ewv):
        @plsc.parallel_loop(0, CE // 16, unroll=4)
        def p2_grp(i):
            ds = pl.ds(i * 16, 16)
            s = es[ds]
            d = ed[ds]
            w = ewv[ds]
            tb = plsc.load_gather(bof, [d])
            plsc.addupdate_scatter(v0, [s], w * tb)
            ti = plsc.load_gather(bif, [s])
            plsc.addupdate_scatter(v1, [d], w * ti)

    edge_pass(p2_proc)

    da = pltpu.async_copy(v0, stage.at[pl.ds(stg(g, 0), NPAD)], sem0)
    db = pltpu.async_copy(v1, stage.at[pl.ds(stg(g, 1), NPAD)], sem1)
    da.wait()
    db.wait()
    plsc.subcore_barrier()

    reduce_round(racc, rbuf, rbufb)

    def mk_a(i, _):
        ds = pl.ds(i * 16, 16)
        ds1 = pl.ds(SLICE + i * 16, 16)
        dego = degsl[ds]
        degi = degsl[ds1]
        rbuf[ds] = racc[ds] / jnp.where(dego > 0.0, dego, 1.0)
        rbuf[ds1] = racc[ds1] / jnp.where(degi > 0.0, degi, 1.0)
        return 0

    lax.fori_loop(0, SLICE // 16, mk_a, 0, unroll=2)

    da = pltpu.async_copy(rbuf.at[pl.ds(0, SLICE)],
                          out.at[pl.ds((bt * 4 + 2) * NPAD + sl0, SLICE)],
                          sem0)
    db = pltpu.async_copy(rbuf.at[pl.ds(SLICE, SLICE)],
                          out.at[pl.ds((bt * 4 + 3) * NPAD + sl0, SLICE)],
                          sem1)
    da.wait()
    db.wait()


_sc_coeffs = pl.kernel(
    _sc_body,
    out_type=jax.ShapeDtypeStruct((B * 4 * NPAD,), jnp.float32),
    mesh=plsc.VectorSubcoreMesh(core_axis_name="c", subcore_axis_name="s"),
    compiler_params=pltpu.CompilerParams(needs_layout_passes=False),
    scratch_types=[
        pltpu.VMEM((CE,), jnp.int32),
        pltpu.VMEM((CE,), jnp.int32),
        pltpu.VMEM((CE,), jnp.float32),
        pltpu.VMEM((CE,), jnp.int32),
        pltpu.VMEM((CE,), jnp.int32),
        pltpu.VMEM((CE,), jnp.float32),
        pltpu.VMEM((NPAD,), jnp.float32),
        pltpu.VMEM((NPAD,), jnp.float32),
        pltpu.VMEM((NPAD,), jnp.float32),
        pltpu.VMEM((NPAD,), jnp.float32),
        pltpu.VMEM((2 * SLICE,), jnp.float32),
        pltpu.VMEM((2 * SLICE,), jnp.float32),
        pltpu.VMEM((2 * SLICE,), jnp.float32),
        pltpu.VMEM((2 * SLICE,), jnp.float32),
        pltpu.VMEM((NPAD,), jnp.float32),
        pltpu.VMEM((NPAD,), jnp.float32),
        pltpu.VMEM_SHARED((2 * GROUPS * 2 * NPAD,), jnp.float32),
        pltpu.VMEM_SHARED((2 * 2 * NPAD,), jnp.float32),
        pltpu.SemaphoreType.DMA,
        pltpu.SemaphoreType.DMA,
    ],
)


def _tc_body(hx, bv, rx, wr0, wrb0, wr1, wrb1, wh0, whb0, wh1, whb1,
             dzw, dzb, dhw, dhb, out, acc, vb0):
    c = pl.program_id(0)

    x = hx[...].reshape(B * CHUNK, 5)
    h = jnp.maximum(
        jnp.dot(x, wh0[...], preferred_element_type=jnp.float32) + whb0[...],
        0.0)
    xh = jnp.maximum(
        jnp.dot(h, wh1[...], preferred_element_type=jnp.float32) + whb1[...],
        0.0)
    bvv = bv[...]

    @pl.when(c == 0)
    def _():
        vb0[...] = jnp.broadcast_to(bvv[:, :, 0:1], (B, 4, XD))

    for b in range(B):
        part = jnp.dot(bvv[b], xh[b * CHUNK:(b + 1) * CHUNK, :],
                       preferred_element_type=jnp.float32)

        @pl.when(c == 0)
        def _():
            acc[b] = part

        @pl.when(c > 0)
        def _():
            acc[b] = acc[b] + part

    @pl.when(c == TC_C - 1)
    def _():
        r = rx[...].reshape(B, 9)
        h0 = jnp.maximum(
            jnp.dot(r, wr0[...], preferred_element_type=jnp.float32)
            + wrb0[...], 0.0)
        x0 = jnp.maximum(
            jnp.dot(h0, wr1[...], preferred_element_type=jnp.float32)
            + wrb1[...], 0.0)
        hz = jnp.maximum(whb0[...], 0.0)
        xh0 = jnp.maximum(
            jnp.dot(hz, wh1[...], preferred_element_type=jnp.float32)
            + whb1[...], 0.0)
        corr = x0 - xh0
        v = acc[...] + vb0[...] * corr.reshape(B, 1, XD)
        v1o = v[:, 0, :]
        v1i = v[:, 1, :]
        v2o = 2.0 * v[:, 2, :] - x0
        v2i = 2.0 * v[:, 3, :] - x0

        def gate(wref, bref):
            w = wref[...]
            pre = (jnp.dot(x0, w[0, 0, :XD, :], preferred_element_type=jnp.float32)
                   + jnp.dot(x0, w[1, 0, :XD, :], preferred_element_type=jnp.float32)
                   + jnp.dot(v1o, w[0, 1, :XD, :], preferred_element_type=jnp.float32)
                   + jnp.dot(v1i, w[1, 1, :XD, :], preferred_element_type=jnp.float32)
                   + jnp.dot(v2o, w[0, 2, :XD, :], preferred_element_type=jnp.float32)
                   + jnp.dot(v2i, w[1, 2, :XD, :], preferred_element_type=jnp.float32))
            return pre + bref[...]

        z = jax.nn.sigmoid(gate(dzw, dzb))
        ht = jnp.tanh(gate(dhw, dhb))
        out[...] = ((1.0 - z) * ht).reshape(B, 1, XD)


def _tc_call(hx_pad, bvecs, robot_x, wr_w0, wr_b0, wr_w1, wr_b1,
             wh_w0, wh_b0, wh_w1, wh_b1, dz_w, dz_b, dh_w, dh_b):
    full = lambda *shape: pl.BlockSpec(shape, lambda c: (0,) * len(shape))
    return pl.pallas_call(
        _tc_body,
        grid=(TC_C,),
        in_specs=[
            pl.BlockSpec((B, CHUNK, 5), lambda c: (0, c, 0)),
            pl.BlockSpec((B, 4, CHUNK), lambda c: (0, 0, c)),
            full(B, 1, 9),
            full(9, 64), full(1, 64), full(64, 32), full(1, 32),
            full(5, 64), full(1, 64), full(64, 32), full(1, 32),
            full(2, 3, 64, 32), full(1, 32),
            full(2, 3, 64, 32), full(1, 32),
        ],
        out_specs=pl.BlockSpec((B, 1, XD), lambda c: (0, 0, 0)),
        out_shape=jax.ShapeDtypeStruct((B, 1, XD), jnp.float32),
        scratch_shapes=[
            pltpu.VMEM((B, 4, XD), jnp.float32),
            pltpu.VMEM((B, 4, XD), jnp.float32),
        ],
    )(hx_pad, bvecs, robot_x, wr_w0, wr_b0, wr_w1, wr_b1,
      wh_w0, wh_b0, wh_w1, wh_b1, dz_w, dz_b, dh_w, dh_b)


@jax.jit
def kernel(robot_x, human_x, edge_index, edge_weight, wr_w0, wr_b0, wr_w1,
           wr_b1, wh_w0, wh_b0, wh_w1, wh_b1, dz_w, dz_b, dr_w, dr_b,
           dh_w, dh_b):
    del dr_w, dr_b
    bvecs = _sc_coeffs(edge_index.reshape(-1),
                       edge_weight.reshape(-1)).reshape(B, 4, NPAD)
    nh = human_x.shape[1]
    hx_pad = jnp.concatenate(
        [jnp.zeros((B, 1, 5), jnp.float32), human_x,
         jnp.zeros((B, NPAD - 1 - nh, 5), jnp.float32)], axis=1)
    res = _tc_call(
        hx_pad, bvecs, robot_x,
        wr_w0, wr_b0.reshape(1, 64), wr_w1, wr_b1.reshape(1, 32),
        wh_w0, wh_b0.reshape(1, 64), wh_w1, wh_b1.reshape(1, 32),
        dz_w, dz_b.reshape(1, 32), dh_w, dh_b.reshape(1, 32))
    return res.reshape(B, XD)


# speedup vs baseline: 437.8399x; 1.1371x over previous
import functools

import jax
import jax.numpy as jnp
from jax import lax
from jax.experimental import pallas as pl
from jax.experimental.pallas import tpu as pltpu
from jax.experimental.pallas import tpu_sc as plsc

B = 4
N = 12500
E = 400000
XD = 32
NPAD = 12800
GROUPS = 8
EPT = E // GROUPS
CE = 2000
NCHUNK = EPT // CE
SLICE = NPAD // GROUPS
CHUNK = 2560
TC_C = NPAD // CHUNK


def _sc_body(ei, ew, out, es0, ed0, ew0, es1, ed1, ew1, v0, v1, v2, v3,
             rbuf, rbufb, racc, degsl, bof, bif, stage, pub, sem0, sem1):
    cid = lax.axis_index("c")
    sid = lax.axis_index("s")
    bt_loc = sid // GROUPS
    g = sid % GROUPS
    bt = cid * 2 + bt_loc
    ebase = g * EPT
    sl0 = g * SLICE
    src_base = bt * 2 * E + ebase
    dst_base = (bt * 2 + 1) * E + ebase
    ew_base = bt * E + ebase

    def stg(j, k):
        return ((bt_loc * GROUPS + j) * 2 + k) * NPAD

    bufs = ((es0, ed0, ew0), (es1, ed1, ew1))
    sems = (sem0, sem1)

    def edge_pass(process_grp):
        def start(ci, pi):
            off = ci * CE
            return (
                pltpu.async_copy(ei.at[pl.ds(src_base + off, CE)],
                                 bufs[pi][0], sems[pi]),
                pltpu.async_copy(ei.at[pl.ds(dst_base + off, CE)],
                                 bufs[pi][1], sems[pi]),
                pltpu.async_copy(ew.at[pl.ds(ew_base + off, CE)],
                                 bufs[pi][2], sems[pi]),
            )

        descs = [start(0, 0), None]
        for ci in range(NCHUNK):
            pi = ci % 2
            if ci + 1 < NCHUNK:
                descs[1 - pi] = start(ci + 1, 1 - pi)
            for dsc in descs[pi]:
                dsc.wait()
            process_grp(*bufs[pi])

    def reduce_round(acc_buf, lbuf_a, lbuf_b):
        def load(j, buf, sem):
            return [pltpu.async_copy(
                stage.at[pl.ds(stg(j, k) + sl0, SLICE)],
                buf.at[pl.ds(k * SLICE, SLICE)], sem) for k in (0, 1)]

        pend = load(0, acc_buf, sem0)
        nxt = load(1, lbuf_a, sem1)
        for d in pend:
            d.wait()
        for j in range(1, GROUPS):
            buf = lbuf_a if j % 2 == 1 else lbuf_b
            upcoming = None
            if j + 1 < GROUPS:
                nbuf = lbuf_a if (j + 1) % 2 == 1 else lbuf_b
                nsem = sem1 if (j + 1) % 2 == 1 else sem0
                upcoming = load(j + 1, nbuf, nsem)
            for d in nxt:
                d.wait()

            @plsc.parallel_loop(0, 2 * SLICE // 16, unroll=4)
            def radd(i):
                ds = pl.ds(i * 16, 16)
                acc_buf[ds] = acc_buf[ds] + buf[ds]
            nxt = upcoming

    zeros16 = jnp.zeros((16,), jnp.float32)

    @plsc.parallel_loop(0, NPAD // 16, unroll=4)
    def zero4(i):
        ds = pl.ds(i * 16, 16)
        v0[ds] = zeros16
        v1[ds] = zeros16
        v2[ds] = zeros16
        v3[ds] = zeros16

    def p1_proc(es, ed, ewv):
        @plsc.parallel_loop(0, CE // 16, unroll=4)
        def p1_grp(i):
            ds = pl.ds(i * 16, 16)
            s = es[ds]
            d = ed[ds]
            w = ewv[ds]
            plsc.addupdate_scatter(v0, [s], w)
            plsc.addupdate_scatter(v1, [d], w)
            plsc.addupdate_scatter(v2, [s], w, mask=(d == 0))
            plsc.addupdate_scatter(v3, [d], w, mask=(s == 0))

    edge_pass(p1_proc)

    da = pltpu.async_copy(v0, stage.at[pl.ds(stg(g, 0), NPAD)], sem0)
    db = pltpu.async_copy(v1, stage.at[pl.ds(stg(g, 1), NPAD)], sem1)
    da.wait()
    db.wait()
    plsc.subcore_barrier()

    reduce_round(degsl, rbuf, rbufb)

    plsc.subcore_barrier()

    da = pltpu.async_copy(v2, stage.at[pl.ds(stg(g, 0), NPAD)], sem0)
    db = pltpu.async_copy(v3, stage.at[pl.ds(stg(g, 1), NPAD)], sem1)
    da.wait()
    db.wait()
    plsc.subcore_barrier()

    reduce_round(racc, rbuf, rbufb)

    @plsc.parallel_loop(0, SLICE // 16, unroll=2)
    def mk_b(i):
        ds = pl.ds(i * 16, 16)
        ds1 = pl.ds(SLICE + i * 16, 16)
        dego = degsl[ds]
        degi = degsl[ds1]
        rbuf[ds] = racc[ds] / jnp.where(dego > 0.0, dego, 1.0)
        rbuf[ds1] = racc[ds1] / jnp.where(degi > 0.0, degi, 1.0)

    ds_ = [
        pltpu.async_copy(rbuf.at[pl.ds(0, SLICE)],
                         out.at[pl.ds((bt * 4 + 0) * NPAD + sl0, SLICE)],
                         sem0),
        pltpu.async_copy(rbuf.at[pl.ds(SLICE, SLICE)],
                         out.at[pl.ds((bt * 4 + 1) * NPAD + sl0, SLICE)],
                         sem0),
        pltpu.async_copy(rbuf.at[pl.ds(0, SLICE)],
                         pub.at[pl.ds(bt_loc * 2 * NPAD + sl0, SLICE)],
                         sem1),
        pltpu.async_copy(rbuf.at[pl.ds(SLICE, SLICE)],
                         pub.at[pl.ds((bt_loc * 2 + 1) * NPAD + sl0, SLICE)],
                         sem1),
    ]
    for d in ds_:
        d.wait()
    plsc.subcore_barrier()

    da = pltpu.async_copy(pub.at[pl.ds(bt_loc * 2 * NPAD, NPAD)], bof, sem0)
    db = pltpu.async_copy(pub.at[pl.ds((bt_loc * 2 + 1) * NPAD, NPAD)], bif,
                          sem1)
    da.wait()
    db.wait()

    @plsc.parallel_loop(0, NPAD // 16, unroll=4)
    def zero2(i):
        ds = pl.ds(i * 16, 16)
        v0[ds] = zeros16
        v1[ds] = zeros16

    def p2_proc(es, ed, ewv):
        @plsc.parallel_loop(0, CE // 16, unroll=4)
        def p2_grp(i):
            ds = pl.ds(i * 16, 16)
            s = es[ds]
            d = ed[ds]
            w = ewv[ds]
            tb = plsc.load_gather(bof, [d])
            plsc.addupdate_scatter(v0, [s], w * tb)
            ti = plsc.load_gather(bif, [s])
            plsc.addupdate_scatter(v1, [d], w * ti)

    edge_pass(p2_proc)

    da = pltpu.async_copy(v0, stage.at[pl.ds(stg(g, 0), NPAD)], sem0)
    db = pltpu.async_copy(v1, stage.at[pl.ds(stg(g, 1), NPAD)], sem1)
    da.wait()
    db.wait()
    plsc.subcore_barrier()

    reduce_round(racc, rbuf, rbufb)

    @plsc.parallel_loop(0, SLICE // 16, unroll=2)
    def mk_a(i):
        ds = pl.ds(i * 16, 16)
        ds1 = pl.ds(SLICE + i * 16, 16)
        dego = degsl[ds]
        degi = degsl[ds1]
        rbuf[ds] = racc[ds] / jnp.where(dego > 0.0, dego, 1.0)
        rbuf[ds1] = racc[ds1] / jnp.where(degi > 0.0, degi, 1.0)

    da = pltpu.async_copy(rbuf.at[pl.ds(0, SLICE)],
                          out.at[pl.ds((bt * 4 + 2) * NPAD + sl0, SLICE)],
                          sem0)
    db = pltpu.async_copy(rbuf.at[pl.ds(SLICE, SLICE)],
                          out.at[pl.ds((bt * 4 + 3) * NPAD + sl0, SLICE)],
                          sem1)
    da.wait()
    db.wait()


_sc_coeffs = pl.kernel(
    _sc_body,
    out_type=jax.ShapeDtypeStruct((B * 4 * NPAD,), jnp.float32),
    mesh=plsc.VectorSubcoreMesh(core_axis_name="c", subcore_axis_name="s"),
    compiler_params=pltpu.CompilerParams(needs_layout_passes=False),
    scratch_types=[
        pltpu.VMEM((CE,), jnp.int32),
        pltpu.VMEM((CE,), jnp.int32),
        pltpu.VMEM((CE,), jnp.float32),
        pltpu.VMEM((CE,), jnp.int32),
        pltpu.VMEM((CE,), jnp.int32),
        pltpu.VMEM((CE,), jnp.float32),
        pltpu.VMEM((NPAD,), jnp.float32),
        pltpu.VMEM((NPAD,), jnp.float32),
        pltpu.VMEM((NPAD,), jnp.float32),
        pltpu.VMEM((NPAD,), jnp.float32),
        pltpu.VMEM((2 * SLICE,), jnp.float32),
        pltpu.VMEM((2 * SLICE,), jnp.float32),
        pltpu.VMEM((2 * SLICE,), jnp.float32),
        pltpu.VMEM((2 * SLICE,), jnp.float32),
        pltpu.VMEM((NPAD,), jnp.float32),
        pltpu.VMEM((NPAD,), jnp.float32),
        pltpu.VMEM_SHARED((2 * GROUPS * 2 * NPAD,), jnp.float32),
        pltpu.VMEM_SHARED((2 * 2 * NPAD,), jnp.float32),
        pltpu.SemaphoreType.DMA,
        pltpu.SemaphoreType.DMA,
    ],
)


def _tc_body(hx, bv, rx, wr0, wrb0, wr1, wrb1, wh0, whb0, wh1, whb1,
             dzw, dzb, dhw, dhb, out, acc, vb0):
    c = pl.program_id(0)

    x = hx[...].reshape(B * CHUNK, 5)
    h = jnp.maximum(
        jnp.dot(x, wh0[...], preferred_element_type=jnp.float32) + whb0[...],
        0.0)
    xh = jnp.maximum(
        jnp.dot(h, wh1[...], preferred_element_type=jnp.float32) + whb1[...],
        0.0)
    bvv = bv[...]

    @pl.when(c == 0)
    def _():
        vb0[...] = jnp.broadcast_to(bvv[:, :, 0:1], (B, 4, XD))

    for b in range(B):
        part = jnp.dot(bvv[b], xh[b * CHUNK:(b + 1) * CHUNK, :],
                       preferred_element_type=jnp.float32)

        @pl.when(c == 0)
        def _():
            acc[b] = part

        @pl.when(c > 0)
        def _():
            acc[b] = acc[b] + part

    @pl.when(c == TC_C - 1)
    def _():
        r = rx[...].reshape(B, 9)
        h0 = jnp.maximum(
            jnp.dot(r, wr0[...], preferred_element_type=jnp.float32)
            + wrb0[...], 0.0)
        x0 = jnp.maximum(
            jnp.dot(h0, wr1[...], preferred_element_type=jnp.float32)
            + wrb1[...], 0.0)
        hz = jnp.maximum(whb0[...], 0.0)
        xh0 = jnp.maximum(
            jnp.dot(hz, wh1[...], preferred_element_type=jnp.float32)
            + whb1[...], 0.0)
        corr = x0 - xh0
        v = acc[...] + vb0[...] * corr.reshape(B, 1, XD)
        v1o = v[:, 0, :]
        v1i = v[:, 1, :]
        v2o = 2.0 * v[:, 2, :] - x0
        v2i = 2.0 * v[:, 3, :] - x0

        def gate(wref, bref):
            w = wref[...]
            pre = (jnp.dot(x0, w[0, 0, :XD, :], preferred_element_type=jnp.float32)
                   + jnp.dot(x0, w[1, 0, :XD, :], preferred_element_type=jnp.float32)
                   + jnp.dot(v1o, w[0, 1, :XD, :], preferred_element_type=jnp.float32)
                   + jnp.dot(v1i, w[1, 1, :XD, :], preferred_element_type=jnp.float32)
                   + jnp.dot(v2o, w[0, 2, :XD, :], preferred_element_type=jnp.float32)
                   + jnp.dot(v2i, w[1, 2, :XD, :], preferred_element_type=jnp.float32))
            return pre + bref[...]

        z = jax.nn.sigmoid(gate(dzw, dzb))
        ht = jnp.tanh(gate(dhw, dhb))
        out[...] = ((1.0 - z) * ht).reshape(B, 1, XD)


def _tc_call(hx_pad, bvecs, robot_x, wr_w0, wr_b0, wr_w1, wr_b1,
             wh_w0, wh_b0, wh_w1, wh_b1, dz_w, dz_b, dh_w, dh_b):
    full = lambda *shape: pl.BlockSpec(shape, lambda c: (0,) * len(shape))
    return pl.pallas_call(
        _tc_body,
        grid=(TC_C,),
        in_specs=[
            pl.BlockSpec((B, CHUNK, 5), lambda c: (0, c, 0)),
            pl.BlockSpec((B, 4, CHUNK), lambda c: (0, 0, c)),
            full(B, 1, 9),
            full(9, 64), full(1, 64), full(64, 32), full(1, 32),
            full(5, 64), full(1, 64), full(64, 32), full(1, 32),
            full(2, 3, 64, 32), full(1, 32),
            full(2, 3, 64, 32), full(1, 32),
        ],
        out_specs=pl.BlockSpec((B, 1, XD), lambda c: (0, 0, 0)),
        out_shape=jax.ShapeDtypeStruct((B, 1, XD), jnp.float32),
        scratch_shapes=[
            pltpu.VMEM((B, 4, XD), jnp.float32),
            pltpu.VMEM((B, 4, XD), jnp.float32),
        ],
    )(hx_pad, bvecs, robot_x, wr_w0, wr_b0, wr_w1, wr_b1,
      wh_w0, wh_b0, wh_w1, wh_b1, dz_w, dz_b, dh_w, dh_b)


@jax.jit
def kernel(robot_x, human_x, edge_index, edge_weight, wr_w0, wr_b0, wr_w1,
           wr_b1, wh_w0, wh_b0, wh_w1, wh_b1, dz_w, dz_b, dr_w, dr_b,
           dh_w, dh_b):
    del dr_w, dr_b
    bvecs = _sc_coeffs(edge_index.reshape(-1),
                       edge_weight.reshape(-1)).reshape(B, 4, NPAD)
    nh = human_x.shape[1]
    hx_pad = jnp.concatenate(
        [jnp.zeros((B, 1, 5), jnp.float32), human_x,
         jnp.zeros((B, NPAD - 1 - nh, 5), jnp.float32)], axis=1)
    res = _tc_call(
        hx_pad, bvecs, robot_x,
        wr_w0, wr_b0.reshape(1, 64), wr_w1, wr_b1.reshape(1, 32),
        wh_w0, wh_b0.reshape(1, 64), wh_w1, wh_b1.reshape(1, 32),
        dz_w, dz_b.reshape(1, 32), dh_w, dh_b.reshape(1, 32))
    return res.reshape(B, XD)
